# baseline (speedup 1.0000x reference)
import functools

import jax
import jax.numpy as jnp
import numpy as np
from jax import lax
from jax.experimental import pallas as pl
from jax.experimental.pallas import tpu as pltpu
from jax.experimental.pallas import tpu_sc as plsc

_VOCAB_SIZE = 32000
_PADDING_IDX = 0
_SMOOTHING = 0.1
_CONFIDENCE = np.float32(1.0 - _SMOOTHING)
_SMOOTH_VAL = np.float32(_SMOOTHING / (_VOCAB_SIZE - 2))

_SUB = 8
_LANE = 128
_CBLK = _VOCAB_SIZE // _LANE
_RB_BLK = 2
_NC = 2
_NS = 16
_LANES = 16


def _fill_body(tgt_ref, out_ref):
    t = tgt_ref[:, 0, :]
    tcol = t[:, None, :, None]
    shape = (_RB_BLK, _CBLK, _SUB, _LANE)
    col = (lax.broadcasted_iota(jnp.int32, shape, 1) * _LANE
           + lax.broadcasted_iota(jnp.int32, shape, 3))
    val = jnp.where((col == _PADDING_IDX) | (tcol == _PADDING_IDX),
                    jnp.float32(0.0), jnp.full((), _SMOOTH_VAL))
    out_ref[...] = val


def _sc_scatter_body(rows_per_sub, tgt_hbm, out_hbm, tgt_v, idx_v, vals_v,
                     sem):
    c = lax.axis_index("c")
    s = lax.axis_index("s")
    wid = s * _NC + c
    base = wid * rows_per_sub
    pltpu.sync_copy(tgt_hbm.at[pl.ds(base, rows_per_sub)], tgt_v)
    lane = lax.iota(jnp.int32, _LANES)
    for g in range(rows_per_sub // _LANES):
        t = tgt_v[pl.ds(g * _LANES, _LANES)]
        row = base + g * _LANES + lane
        idx = (((row >> 3) * _CBLK + (t >> 7)) << 10) \
            + ((row & 7) << 7) + (t & 127)
        idx_v[pl.ds(g * _LANES, _LANES)] = idx
        vals_v[pl.ds(g * _LANES, _LANES)] = jnp.where(
            t == _PADDING_IDX, jnp.float32(0.0),
            jnp.full((), _CONFIDENCE))
    pltpu.async_copy(vals_v, out_hbm.at[idx_v], sem).wait()


@jax.jit
def kernel(targets):
    batch_size, tgt_seq_len = targets.shape
    rows = batch_size * tgt_seq_len
    num_rb = rows // _SUB
    num_blocks = num_rb // _RB_BLK
    tgt_r = targets.reshape(num_rb, 1, _SUB)

    filled = pl.pallas_call(
        _fill_body,
        grid=(num_blocks,),
        in_specs=[pl.BlockSpec((_RB_BLK, 1, _SUB), lambda i: (i, 0, 0))],
        out_specs=pl.BlockSpec((_RB_BLK, _CBLK, _SUB, _LANE),
                               lambda i: (i, 0, 0, 0)),
        out_shape=jax.ShapeDtypeStruct((num_rb, _CBLK, _SUB, _LANE),
                                       jnp.float32),
    )(tgt_r)

    rows_per_sub = rows // (_NC * _NS)
    out_ref = jax.new_ref(filled.reshape(rows * _VOCAB_SIZE))
    mesh = plsc.VectorSubcoreMesh(
        core_axis_name="c", subcore_axis_name="s",
        num_cores=_NC, num_subcores=_NS)
    sc_scatter = pl.kernel(
        functools.partial(_sc_scatter_body, rows_per_sub),
        out_type=(),
        mesh=mesh,
        scratch_types=[
            pltpu.VMEM((rows_per_sub,), jnp.int32),
            pltpu.VMEM((rows_per_sub,), jnp.int32),
            pltpu.VMEM((rows_per_sub,), jnp.float32),
            pltpu.SemaphoreType.DMA,
        ],
    )
    sc_scatter(targets.reshape(rows), out_ref)
    out = out_ref[...].reshape(num_rb, _CBLK, _SUB, _LANE)
    out = out.transpose(0, 2, 1, 3).reshape(batch_size, tgt_seq_len,
                                            _VOCAB_SIZE)
    return out


# speedup vs baseline: 1.2658x; 1.2658x over previous
import functools

import jax
import jax.numpy as jnp
import numpy as np
from jax import lax
from jax.experimental import pallas as pl
from jax.experimental.pallas import tpu as pltpu
from jax.experimental.pallas import tpu_sc as plsc

_VOCAB_SIZE = 32000
_PADDING_IDX = 0
_SMOOTHING = 0.1
_CONFIDENCE = np.float32(1.0 - _SMOOTHING)
_SMOOTH_VAL = np.float32(_SMOOTHING / (_VOCAB_SIZE - 2))

_SUB = 8
_LANE = 128
_CBLK = _VOCAB_SIZE // _LANE
_RB_BLK = 4
_NC = 2
_NS = 16
_LANES = 16


def _fill_body(tgt_ref, out_ref):
    t = tgt_ref[:, 0, :]
    tcol = t[:, None, :, None]
    shape = (_RB_BLK, _CBLK, _SUB, _LANE)
    col = (lax.broadcasted_iota(jnp.int32, shape, 1) * _LANE
           + lax.broadcasted_iota(jnp.int32, shape, 3))
    val = jnp.where((col == _PADDING_IDX) | (tcol == _PADDING_IDX),
                    jnp.float32(0.0), jnp.full((), _SMOOTH_VAL))
    out_ref[...] = val


def _sc_scatter_body(rows_per_sub, tgt_hbm, out_hbm, tgt_v, idx_v, vals_v,
                     sem):
    c = lax.axis_index("c")
    s = lax.axis_index("s")
    wid = s * _NC + c
    base = wid * rows_per_sub
    pltpu.sync_copy(tgt_hbm.at[pl.ds(base, rows_per_sub)], tgt_v)
    lane = lax.iota(jnp.int32, _LANES)
    for g in range(rows_per_sub // _LANES):
        t = tgt_v[pl.ds(g * _LANES, _LANES)]
        row = base + g * _LANES + lane
        idx = (((row >> 3) * _CBLK + (t >> 7)) << 10) \
            + ((row & 7) << 7) + (t & 127)
        idx_v[pl.ds(g * _LANES, _LANES)] = idx
        vals_v[pl.ds(g * _LANES, _LANES)] = jnp.where(
            t == _PADDING_IDX, jnp.float32(0.0),
            jnp.full((), _CONFIDENCE))
    pltpu.async_copy(vals_v, out_hbm.at[idx_v], sem).wait()


@jax.jit
def kernel(targets):
    batch_size, tgt_seq_len = targets.shape
    rows = batch_size * tgt_seq_len
    num_rb = rows // _SUB
    num_blocks = num_rb // _RB_BLK
    tgt_r = targets.reshape(num_rb, 1, _SUB)

    filled = pl.pallas_call(
        _fill_body,
        grid=(num_blocks,),
        in_specs=[pl.BlockSpec((_RB_BLK, 1, _SUB), lambda i: (i, 0, 0))],
        out_specs=pl.BlockSpec((_RB_BLK, _CBLK, _SUB, _LANE),
                               lambda i: (i, 0, 0, 0)),
        out_shape=jax.ShapeDtypeStruct((num_rb, _CBLK, _SUB, _LANE),
                                       jnp.float32),
    )(tgt_r)

    rows_per_sub = rows // (_NC * _NS)
    out_ref = jax.new_ref(filled.reshape(rows * _VOCAB_SIZE))
    mesh = plsc.VectorSubcoreMesh(
        core_axis_name="c", subcore_axis_name="s",
        num_cores=_NC, num_subcores=_NS)
    sc_scatter = pl.kernel(
        functools.partial(_sc_scatter_body, rows_per_sub),
        out_type=(),
        mesh=mesh,
        scratch_types=[
            pltpu.VMEM((rows_per_sub,), jnp.int32),
            pltpu.VMEM((rows_per_sub,), jnp.int32),
            pltpu.VMEM((rows_per_sub,), jnp.float32),
            pltpu.SemaphoreType.DMA,
        ],
    )
    sc_scatter(targets.reshape(rows), out_ref)
    out = out_ref[...].reshape(num_rb, _CBLK, _SUB, _LANE)
    out = out.transpose(0, 2, 1, 3).reshape(batch_size, tgt_seq_len,
                                            _VOCAB_SIZE)
    return out
